# baseline (speedup 1.0000x reference)
import functools

import jax
import jax.numpy as jnp
from jax.experimental import pallas as pl
from jax.experimental.pallas import tpu as pltpu

_BIG = 100000000.0


def _cost_kernel(logits_ref, pb_ref, boxes_ref, lab_ref, area_ref, out_ref,
                 bt_ref):
    b = pl.program_id(0)

    @pl.when(pl.program_id(1) == 0)
    def _():
        bt_ref[0:4, :] = jnp.transpose(boxes_ref[b], (1, 0))

    logits = logits_ref[0]
    m = jnp.max(logits, axis=-1, keepdims=True)
    e = jnp.exp(logits - m)
    p = e / jnp.sum(e, axis=-1, keepdims=True)

    labels = lab_ref[pl.ds(b, 1), :]
    c = logits.shape[-1]
    q = labels.shape[-1]
    cls = jax.lax.broadcasted_iota(jnp.int32, (c, q), 0)
    onehot = (labels == cls).astype(jnp.float32)
    cost_class = -jax.lax.dot_general(
        p, onehot, (((1,), (0,)), ((), ())),
        preferred_element_type=jnp.float32)

    pb = pb_ref[0]
    cxp, cyp = pb[:, 0:1], pb[:, 1:2]
    wp, hp = pb[:, 2:3], pb[:, 3:4]
    cxb, cyb = bt_ref[0:1, :], bt_ref[1:2, :]
    wb, hb = bt_ref[2:3, :], bt_ref[3:4, :]

    cost_bbox = 0.25 * (jnp.abs(cxp - cxb) + jnp.abs(cyp - cyb)
                        + jnp.abs(wp - wb) + jnp.abs(hp - hb))

    x0p, x1p = cxp - 0.5 * wp, cxp + 0.5 * wp
    y0p, y1p = cyp - 0.5 * hp, cyp + 0.5 * hp
    x0b, x1b = cxb - 0.5 * wb, cxb + 0.5 * wb
    y0b, y1b = cyb - 0.5 * hb, cyb + 0.5 * hb
    a1 = (x1p - x0p) * (y1p - y0p)
    a2 = (x1b - x0b) * (y1b - y0b)

    wx = jnp.maximum(jnp.minimum(x1p, x1b) - jnp.maximum(x0p, x0b), 0.0)
    wy = jnp.maximum(jnp.minimum(y1p, y1b) - jnp.maximum(y0p, y0b), 0.0)
    inter = wx * wy
    union = (a1 + a2) - inter
    iou = inter / union

    wex = jnp.maximum(x1p, x1b) - jnp.minimum(x0p, x0b)
    wey = jnp.maximum(y1p, y1b) - jnp.minimum(y0p, y0b)
    enc = wex * wey
    cost = cost_bbox + cost_class - iou + (enc - union) / enc

    mask = area_ref[pl.ds(b, 1), :] > 0.0
    out_ref[0] = jnp.where(mask, cost, _BIG)


@jax.jit
def kernel(pred_logits, pred_boxes, boxes, area, labels):
    b, q, c = pred_logits.shape
    bi = 128
    n_i = pl.cdiv(q, bi)

    return pl.pallas_call(
        _cost_kernel,
        grid=(b, n_i),
        in_specs=[
            pl.BlockSpec((1, bi, c), lambda ib, ii: (ib, ii, 0)),
            pl.BlockSpec((1, bi, 4), lambda ib, ii: (ib, ii, 0)),
            pl.BlockSpec((b, q, 4), lambda ib, ii: (0, 0, 0)),
            pl.BlockSpec((b, q), lambda ib, ii: (0, 0)),
            pl.BlockSpec((b, q), lambda ib, ii: (0, 0)),
        ],
        out_specs=pl.BlockSpec((1, bi, q), lambda ib, ii: (ib, ii, 0)),
        out_shape=jax.ShapeDtypeStruct((b, q, q), jnp.float32),
        scratch_shapes=[pltpu.VMEM((8, q), jnp.float32)],
        compiler_params=pltpu.CompilerParams(
            dimension_semantics=("parallel", "arbitrary"),
        ),
        name="hungarian_cost",
    )(pred_logits, pred_boxes, boxes, labels.astype(jnp.int32), area)


# speedup vs baseline: 1.0008x; 1.0008x over previous
import functools

import jax
import jax.numpy as jnp
from jax.experimental import pallas as pl
from jax.experimental.pallas import tpu as pltpu
from jax.experimental.layout import Layout as _Layout
from jax.experimental.layout import with_layout_constraint as _with_layout

_BIG = 100000000.0


def _cost_kernel(logits_ref, pb_ref, boxes_ref, lab_ref, area_ref, out_ref,
                 bt_ref):
    b = pl.program_id(0)

    @pl.when(pl.program_id(1) == 0)
    def _():
        bt_ref[0:4, :] = jnp.transpose(boxes_ref[b], (1, 0))

    logits = logits_ref[0]
    m = jnp.max(logits, axis=-1, keepdims=True)
    e = jnp.exp(logits - m)
    p = e / jnp.sum(e, axis=-1, keepdims=True)

    labels = lab_ref[pl.ds(b, 1), :]
    c = logits.shape[-1]
    q = labels.shape[-1]
    cls = jax.lax.broadcasted_iota(jnp.int32, (c, q), 0)
    onehot = (labels == cls).astype(jnp.float32)
    cost_class = -jax.lax.dot_general(
        p, onehot, (((1,), (0,)), ((), ())),
        preferred_element_type=jnp.float32)

    pb = pb_ref[0]
    cxp, cyp = pb[:, 0:1], pb[:, 1:2]
    wp, hp = pb[:, 2:3], pb[:, 3:4]
    cxb, cyb = bt_ref[0:1, :], bt_ref[1:2, :]
    wb, hb = bt_ref[2:3, :], bt_ref[3:4, :]

    cost_bbox = 0.25 * (jnp.abs(cxp - cxb) + jnp.abs(cyp - cyb)
                        + jnp.abs(wp - wb) + jnp.abs(hp - hb))

    x0p, x1p = cxp - 0.5 * wp, cxp + 0.5 * wp
    y0p, y1p = cyp - 0.5 * hp, cyp + 0.5 * hp
    x0b, x1b = cxb - 0.5 * wb, cxb + 0.5 * wb
    y0b, y1b = cyb - 0.5 * hb, cyb + 0.5 * hb
    a1 = (x1p - x0p) * (y1p - y0p)
    a2 = (x1b - x0b) * (y1b - y0b)

    wx = jnp.maximum(jnp.minimum(x1p, x1b) - jnp.maximum(x0p, x0b), 0.0)
    wy = jnp.maximum(jnp.minimum(y1p, y1b) - jnp.maximum(y0p, y0b), 0.0)
    inter = wx * wy
    union = (a1 + a2) - inter
    iou = inter / union

    wex = jnp.maximum(x1p, x1b) - jnp.minimum(x0p, x0b)
    wey = jnp.maximum(y1p, y1b) - jnp.minimum(y0p, y0b)
    enc = wex * wey
    cost = cost_bbox + cost_class - iou + (enc - union) / enc

    mask = area_ref[pl.ds(b, 1), :] > 0.0
    out_ref[0] = jnp.where(mask, cost, _BIG)


@jax.jit
def kernel(pred_logits, pred_boxes, boxes, area, labels):
    b, q, c = pred_logits.shape
    bi = 128
    n_i = pl.cdiv(q, bi)

    out = pl.pallas_call(
        _cost_kernel,
        grid=(b, n_i),
        in_specs=[
            pl.BlockSpec((1, bi, c), lambda ib, ii: (ib, ii, 0)),
            pl.BlockSpec((1, bi, 4), lambda ib, ii: (ib, ii, 0)),
            pl.BlockSpec((b, q, 4), lambda ib, ii: (0, 0, 0)),
            pl.BlockSpec((b, q), lambda ib, ii: (0, 0)),
            pl.BlockSpec((b, q), lambda ib, ii: (0, 0)),
        ],
        out_specs=pl.BlockSpec((1, bi, q), lambda ib, ii: (ib, ii, 0)),
        out_shape=jax.ShapeDtypeStruct((b, q, q), jnp.float32),
        scratch_shapes=[pltpu.VMEM((8, q), jnp.float32)],
        compiler_params=pltpu.CompilerParams(
            dimension_semantics=("parallel", "arbitrary"),
        ),
        name="hungarian_cost",
    )(pred_logits, pred_boxes, boxes, labels.astype(jnp.int32), area)
    return _with_layout(out, _Layout(major_to_minor=(0, 1, 2)))


# speedup vs baseline: 1.4348x; 1.4336x over previous
import functools

import jax
import jax.numpy as jnp
from jax.experimental import pallas as pl
from jax.experimental.pallas import tpu as pltpu
from jax.experimental.layout import Layout as _Layout
from jax.experimental.layout import with_layout_constraint as _with_layout

_BIG = 100000000.0


def _cost_kernel(logits_ref, pb_ref, boxes_ref, lab_ref, area_ref, out_ref,
                 bt_ref):
    b = pl.program_id(0)

    @pl.when(pl.program_id(1) == 0)
    def _():
        bt_ref[0:4, :] = jnp.transpose(boxes_ref[b], (1, 0))

    logits = logits_ref[0]
    m = jnp.max(logits, axis=-1, keepdims=True)
    e = jnp.exp(logits - m)
    p = e / jnp.sum(e, axis=-1, keepdims=True)

    labels = lab_ref[pl.ds(b, 1), :]
    c = logits.shape[-1]
    q = labels.shape[-1]
    cls = jax.lax.broadcasted_iota(jnp.int32, (c, q), 0)
    onehot = (labels == cls).astype(jnp.float32)
    cost_class = -jax.lax.dot_general(
        p, onehot, (((1,), (0,)), ((), ())),
        preferred_element_type=jnp.float32)

    pb = pb_ref[0]
    cxp, cyp = pb[:, 0:1], pb[:, 1:2]
    wp, hp = pb[:, 2:3], pb[:, 3:4]
    cxb, cyb = bt_ref[0:1, :], bt_ref[1:2, :]
    wb, hb = bt_ref[2:3, :], bt_ref[3:4, :]

    cost_bbox = 0.25 * (jnp.abs(cxp - cxb) + jnp.abs(cyp - cyb)
                        + jnp.abs(wp - wb) + jnp.abs(hp - hb))

    x0p, x1p = cxp - 0.5 * wp, cxp + 0.5 * wp
    y0p, y1p = cyp - 0.5 * hp, cyp + 0.5 * hp
    x0b, x1b = cxb - 0.5 * wb, cxb + 0.5 * wb
    y0b, y1b = cyb - 0.5 * hb, cyb + 0.5 * hb
    a1 = (x1p - x0p) * (y1p - y0p)
    a2 = (x1b - x0b) * (y1b - y0b)

    wx = jnp.maximum(jnp.minimum(x1p, x1b) - jnp.maximum(x0p, x0b), 0.0)
    wy = jnp.maximum(jnp.minimum(y1p, y1b) - jnp.maximum(y0p, y0b), 0.0)
    inter = wx * wy
    union = (a1 + a2) - inter
    iou = inter / union

    wex = jnp.maximum(x1p, x1b) - jnp.minimum(x0p, x0b)
    wey = jnp.maximum(y1p, y1b) - jnp.minimum(y0p, y0b)
    enc = wex * wey
    cost = cost_bbox + cost_class - iou + (enc - union) / enc

    mask = area_ref[pl.ds(b, 1), :] > 0.0
    out_ref[0] = jnp.where(mask, cost, _BIG)


def kernel(pred_logits, pred_boxes, boxes, area, labels):
    b, q, c = pred_logits.shape
    bi = 128
    n_i = pl.cdiv(q, bi)

    out = pl.pallas_call(
        _cost_kernel,
        grid=(b, n_i),
        in_specs=[
            pl.BlockSpec((1, bi, c), lambda ib, ii: (ib, ii, 0)),
            pl.BlockSpec((1, bi, 4), lambda ib, ii: (ib, ii, 0)),
            pl.BlockSpec((b, q, 4), lambda ib, ii: (0, 0, 0)),
            pl.BlockSpec((b, q), lambda ib, ii: (0, 0)),
            pl.BlockSpec((b, q), lambda ib, ii: (0, 0)),
        ],
        out_specs=pl.BlockSpec((1, bi, q), lambda ib, ii: (ib, ii, 0)),
        out_shape=jax.ShapeDtypeStruct((b, q, q), jnp.float32),
        scratch_shapes=[pltpu.VMEM((8, q), jnp.float32)],
        compiler_params=pltpu.CompilerParams(
            dimension_semantics=("parallel", "arbitrary"),
        ),
        name="hungarian_cost",
    )(pred_logits, pred_boxes, boxes, labels.astype(jnp.int32), area)
    return _with_layout(out, _Layout(major_to_minor=(0, 1, 2)))
